# baseline (speedup 1.0000x reference)
import functools
import math

import jax
import jax.numpy as jnp
from jax.experimental import pallas as pl


def _g_body(oht_ref, wct_ref, g_ref):
    g_ref[...] = jax.lax.dot_general(
        oht_ref[...], wct_ref[...],
        dimension_numbers=(((0,), (0,)), ((), ())),
        preferred_element_type=jnp.float32)


def _pool_body(f_ref, g_ref, pooled_ref, *, keep_cnt, inv_hw):
    ft = f_ref[...]
    g = g_ref[...]
    sm = jnp.sum(ft * g[None, :, :], axis=2)
    smt = jnp.transpose(sm)
    hw = sm.shape[0]
    cnt = jnp.zeros_like(smt)
    for j in range(hw):
        col_j = jnp.broadcast_to(smt[:, j:j + 1], smt.shape)
        cnt += (col_j >= smt).astype(jnp.float32)
    mask = jnp.transpose((cnt >= keep_cnt).astype(jnp.float32))
    pooled_ref[...] = jnp.sum(ft * mask[:, :, None], axis=0) * inv_hw


def _softmax0(x):
    m = jnp.max(x, axis=0, keepdims=True)
    e = jnp.exp(x - m)
    return e / jnp.sum(e, axis=0, keepdims=True)


def _loss_body(pooled_ref, wct_ref, bct_ref, predst_ref, oht_ref, out_ref,
               *, n_rows, th_idx, eps):
    pooled = pooled_ref[...]
    logits_t = jax.lax.dot_general(
        wct_ref[...], pooled,
        dimension_numbers=(((1,), (1,)), ((), ())),
        preferred_element_type=jnp.float32) + bct_ref[...]
    preds_t = predst_ref[...]
    oh_t = oht_ref[...]
    before = jnp.sum(oh_t * _softmax0(preds_t), axis=0, keepdims=True)
    after = jnp.sum(oh_t * _softmax0(logits_t), axis=0, keepdims=True)
    change = jnp.maximum(before - after - eps, 0.0)
    change_col = jnp.transpose(change)
    cmp = (jnp.broadcast_to(change_col, (n_rows, n_rows)) <
           jnp.broadcast_to(change, (n_rows, n_rows)))
    cnt_lt = jnp.sum(cmp.astype(jnp.float32), axis=0, keepdims=True)
    drop = (cnt_lt >= float(n_rows - th_idx)).astype(jnp.float32)
    logits = drop * logits_t + (1.0 - drop) * preds_t
    m = jnp.max(logits, axis=0, keepdims=True)
    lse = jnp.log(jnp.sum(jnp.exp(logits - m), axis=0, keepdims=True)) + m
    out_ref[...] = -jnp.sum(oh_t * (logits - lse), axis=(0, 1),
                            keepdims=True) / float(n_rows)


@jax.jit
def kernel(features, preds, labels, oh_labels, Wc, bc):
    N, C, H, W = features.shape
    HW = H * W
    K = preds.shape[1]
    ft = jnp.transpose(features, (2, 3, 0, 1)).reshape(HW, N, C)
    oh_t = jnp.transpose(oh_labels)
    wc_t = jnp.transpose(Wc)
    preds_t = jnp.transpose(preds)

    G = pl.pallas_call(
        _g_body,
        out_shape=jax.ShapeDtypeStruct((N, C), jnp.float32),
    )(oh_t, wc_t)

    B = 32
    keep_cnt = float(math.ceil(HW / 3.0) + 1)
    pooled = pl.pallas_call(
        functools.partial(_pool_body, keep_cnt=keep_cnt, inv_hw=1.0 / HW),
        grid=(N // B,),
        in_specs=[
            pl.BlockSpec((HW, B, C), lambda i: (0, i, 0)),
            pl.BlockSpec((B, C), lambda i: (i, 0)),
        ],
        out_specs=pl.BlockSpec((B, C), lambda i: (i, 0)),
        out_shape=jax.ShapeDtypeStruct((N, C), jnp.float32),
    )(ft, G)

    th_idx = int(round(float(N) * 0.3333))
    loss = pl.pallas_call(
        functools.partial(_loss_body, n_rows=N, th_idx=th_idx, eps=1e-4),
        out_shape=jax.ShapeDtypeStruct((1, 1), jnp.float32),
    )(pooled, wc_t, bc.reshape(K, 1), preds_t, oh_t)
    return loss[0, 0]


# speedup vs baseline: 1.0126x; 1.0126x over previous
import functools
import math

import jax
import jax.numpy as jnp
from jax.experimental import pallas as pl


def _g_body(oht_ref, wct_ref, g_ref):
    g_ref[...] = jax.lax.dot_general(
        oht_ref[...], wct_ref[...],
        dimension_numbers=(((0,), (0,)), ((), ())),
        preferred_element_type=jnp.float32)


def _pool_body(f_ref, g_ref, pooled_ref, *, keep_cnt, inv_hw):
    ft = f_ref[...]
    g = g_ref[...]
    sm = jnp.sum(ft * g[None, :, :], axis=2)
    smt = jnp.transpose(sm)
    hw = sm.shape[0]
    cnt = jnp.zeros_like(smt)
    for j in range(hw):
        col_j = jnp.broadcast_to(smt[:, j:j + 1], smt.shape)
        cnt += (col_j >= smt).astype(jnp.float32)
    mask = jnp.transpose((cnt >= keep_cnt).astype(jnp.float32))
    pooled_ref[...] = jnp.sum(ft * mask[:, :, None], axis=0) * inv_hw


def _softmax0(x):
    m = jnp.max(x, axis=0, keepdims=True)
    e = jnp.exp(x - m)
    return e / jnp.sum(e, axis=0, keepdims=True)


def _loss_body(pooled_ref, wct_ref, bct_ref, predst_ref, oht_ref, out_ref,
               *, n_rows, th_idx, eps):
    pooled = pooled_ref[...]
    logits_t = jax.lax.dot_general(
        wct_ref[...], pooled,
        dimension_numbers=(((1,), (1,)), ((), ())),
        preferred_element_type=jnp.float32) + bct_ref[...]
    preds_t = predst_ref[...]
    oh_t = oht_ref[...]
    before = jnp.sum(oh_t * _softmax0(preds_t), axis=0, keepdims=True)
    after = jnp.sum(oh_t * _softmax0(logits_t), axis=0, keepdims=True)
    change = jnp.maximum(before - after - eps, 0.0)
    change_col = jnp.transpose(change)
    cmp = (jnp.broadcast_to(change_col, (n_rows, n_rows)) <
           jnp.broadcast_to(change, (n_rows, n_rows)))
    cnt_lt = jnp.sum(cmp.astype(jnp.float32), axis=0, keepdims=True)
    drop = (cnt_lt >= float(n_rows - th_idx)).astype(jnp.float32)
    logits = drop * logits_t + (1.0 - drop) * preds_t
    m = jnp.max(logits, axis=0, keepdims=True)
    lse = jnp.log(jnp.sum(jnp.exp(logits - m), axis=0, keepdims=True)) + m
    out_ref[...] = -jnp.sum(oh_t * (logits - lse), axis=(0, 1),
                            keepdims=True) / float(n_rows)


@jax.jit
def kernel(features, preds, labels, oh_labels, Wc, bc):
    N, C, H, W = features.shape
    HW = H * W
    K = preds.shape[1]
    ft = jnp.transpose(features, (2, 3, 0, 1)).reshape(HW, N, C)
    oh_t = jnp.transpose(oh_labels)
    wc_t = jnp.transpose(Wc)
    preds_t = jnp.transpose(preds)

    G = pl.pallas_call(
        _g_body,
        out_shape=jax.ShapeDtypeStruct((N, C), jnp.float32),
    )(oh_t, wc_t)

    B = 64
    keep_cnt = float(math.ceil(HW / 3.0) + 1)
    pooled = pl.pallas_call(
        functools.partial(_pool_body, keep_cnt=keep_cnt, inv_hw=1.0 / HW),
        grid=(N // B,),
        in_specs=[
            pl.BlockSpec((HW, B, C), lambda i: (0, i, 0)),
            pl.BlockSpec((B, C), lambda i: (i, 0)),
        ],
        out_specs=pl.BlockSpec((B, C), lambda i: (i, 0)),
        out_shape=jax.ShapeDtypeStruct((N, C), jnp.float32),
    )(ft, G)

    th_idx = int(round(float(N) * 0.3333))
    loss = pl.pallas_call(
        functools.partial(_loss_body, n_rows=N, th_idx=th_idx, eps=1e-4),
        out_shape=jax.ShapeDtypeStruct((1, 1), jnp.float32),
    )(pooled, wc_t, bc.reshape(K, 1), preds_t, oh_t)
    return loss[0, 0]


# speedup vs baseline: 1.0387x; 1.0257x over previous
import functools
import math

import jax
import jax.numpy as jnp
from jax.experimental import pallas as pl
from jax.experimental.pallas import tpu as pltpu


def _pool_body(oht_ref, wct_ref, f_ref, pooled_ref, g_scr, *, b, keep_cnt,
               inv_hw):
    @pl.when(pl.program_id(0) == 0)
    def _():
        g_scr[...] = jax.lax.dot_general(
            oht_ref[...], wct_ref[...],
            dimension_numbers=(((0,), (0,)), ((), ())),
            preferred_element_type=jnp.float32)

    ft = f_ref[...]
    i = pl.program_id(0)
    g = g_scr[pl.ds(pl.multiple_of(i * b, b), b), :]
    sm = jnp.sum(ft * g[None, :, :], axis=2)
    smt = jnp.transpose(sm)
    hw = sm.shape[0]
    cnt = jnp.zeros_like(smt)
    for j in range(hw):
        col_j = jnp.broadcast_to(smt[:, j:j + 1], smt.shape)
        cnt += (col_j >= smt).astype(jnp.float32)
    mask = jnp.transpose((cnt >= keep_cnt).astype(jnp.float32))
    pooled_ref[...] = jnp.sum(ft * mask[:, :, None], axis=0) * inv_hw


def _softmax0(x):
    m = jnp.max(x, axis=0, keepdims=True)
    e = jnp.exp(x - m)
    return e / jnp.sum(e, axis=0, keepdims=True)


def _loss_body(pooled_ref, wct_ref, bct_ref, predst_ref, oht_ref, out_ref,
               *, n_rows, th_idx, eps):
    pooled = pooled_ref[...]
    logits_t = jax.lax.dot_general(
        wct_ref[...], pooled,
        dimension_numbers=(((1,), (1,)), ((), ())),
        preferred_element_type=jnp.float32) + bct_ref[...]
    preds_t = predst_ref[...]
    oh_t = oht_ref[...]
    before = jnp.sum(oh_t * _softmax0(preds_t), axis=0, keepdims=True)
    after = jnp.sum(oh_t * _softmax0(logits_t), axis=0, keepdims=True)
    change = jnp.maximum(before - after - eps, 0.0)
    change_col = jnp.transpose(change)
    cmp = (jnp.broadcast_to(change_col, (n_rows, n_rows)) <
           jnp.broadcast_to(change, (n_rows, n_rows)))
    cnt_lt = jnp.sum(cmp.astype(jnp.float32), axis=0, keepdims=True)
    drop = (cnt_lt >= float(n_rows - th_idx)).astype(jnp.float32)
    logits = drop * logits_t + (1.0 - drop) * preds_t
    m = jnp.max(logits, axis=0, keepdims=True)
    lse = jnp.log(jnp.sum(jnp.exp(logits - m), axis=0, keepdims=True)) + m
    out_ref[...] = -jnp.sum(oh_t * (logits - lse), axis=(0, 1),
                            keepdims=True) / float(n_rows)


@jax.jit
def kernel(features, preds, labels, oh_labels, Wc, bc):
    N, C, H, W = features.shape
    HW = H * W
    K = preds.shape[1]
    ft = jnp.transpose(features, (2, 3, 0, 1)).reshape(HW, N, C)
    oh_t = jnp.transpose(oh_labels)
    wc_t = jnp.transpose(Wc)
    preds_t = jnp.transpose(preds)

    B = 32
    keep_cnt = float(math.ceil(HW / 3.0) + 1)
    pooled = pl.pallas_call(
        functools.partial(_pool_body, b=B, keep_cnt=keep_cnt,
                          inv_hw=1.0 / HW),
        grid=(N // B,),
        in_specs=[
            pl.BlockSpec((K, N), lambda i: (0, 0)),
            pl.BlockSpec((K, C), lambda i: (0, 0)),
            pl.BlockSpec((HW, B, C), lambda i: (0, i, 0)),
        ],
        out_specs=pl.BlockSpec((B, C), lambda i: (i, 0)),
        out_shape=jax.ShapeDtypeStruct((N, C), jnp.float32),
        scratch_shapes=[pltpu.VMEM((N, C), jnp.float32)],
    )(oh_t, wc_t, ft)

    th_idx = int(round(float(N) * 0.3333))
    loss = pl.pallas_call(
        functools.partial(_loss_body, n_rows=N, th_idx=th_idx, eps=1e-4),
        out_shape=jax.ShapeDtypeStruct((1, 1), jnp.float32),
    )(pooled, wc_t, bc.reshape(K, 1), preds_t, oh_t)
    return loss[0, 0]
